# baseline (speedup 1.0000x reference)
import functools

import jax
import jax.numpy as jnp
from jax.experimental import pallas as pl
from jax.experimental.pallas import tpu as pltpu

_NEG_SLOPE = 0.01
_NUM_CLASSES = 10
_NB = 16


def _lrelu(v):
    return jnp.where(v >= 0.0, v, _NEG_SLOPE * v)


def _fused_body(x_ref, w1a_ref, w1b_ref, b1_ref, w2_ref, b2_ref,
                wf1_ref, bf1_ref, bns_ref, bnt_ref, wf2_ref, bf2_ref,
                o_ref, *, D, H, WC, NB, half1, half2):
    f32 = jnp.float32
    Do1, Ho1 = D - 2, H - 2
    S2 = Do1 // 2
    Do2 = S2 - 2
    Dp2 = Do2 // 2

    xb = x_ref[0].astype(jnp.bfloat16)

    pad = 128 - 3 * WC
    pz = jnp.zeros((D, Ho1, NB, pad), jnp.bfloat16)
    P = jnp.concatenate(
        [xb[:, 0:Ho1], xb[:, 1:1 + Ho1], xb[:, 2:2 + Ho1], pz], axis=-1)

    lhs1 = jnp.concatenate([P[0:Do1], P[1:1 + Do1], P[2:2 + Do1]], axis=-1)
    lhs1 = lhs1.reshape(Do1 * Ho1 * NB, 384)
    q = jnp.maximum(
        jnp.dot(lhs1, w1a_ref[...], preferred_element_type=f32),
        jnp.dot(lhs1, w1b_ref[...], preferred_element_type=f32))
    q = q.reshape(S2, 2, Ho1, NB, half1)
    q = jnp.maximum(q[:, 0], q[:, 1])
    q = q.reshape(S2, S2, 2, NB, half1)
    q = jnp.maximum(q[:, :, 0], q[:, :, 1])
    p1 = _lrelu(q + b1_ref[...]).astype(jnp.bfloat16)

    pieces = [p1[kd:kd + Do2, kh:kh + Do2] for kd in range(3) for kh in range(3)]
    l2 = jnp.concatenate(pieces, axis=-1).reshape(Do2 * Do2 * NB, 9 * half1)
    r2 = jnp.dot(l2, w2_ref[...], preferred_element_type=f32)

    q2 = jnp.maximum(r2[:, :half2], r2[:, half2:])
    q2 = q2.reshape(Dp2, 2, Do2, NB, half2)
    q2 = jnp.maximum(q2[:, 0], q2[:, 1])
    q2 = q2.reshape(Dp2, Dp2, 2, NB, half2)
    q2 = jnp.maximum(q2[:, :, 0], q2[:, :, 1])
    p2 = _lrelu(q2 + b2_ref[...]).astype(jnp.bfloat16)

    fz = jnp.concatenate([p2[0, 0], p2[0, 1], p2[1, 0], p2[1, 1]], axis=-1)
    h = jnp.dot(fz, wf1_ref[...], preferred_element_type=f32) + bf1_ref[...]
    h = _lrelu(h) * bns_ref[...] + bnt_ref[...]
    o = jnp.dot(h.astype(jnp.bfloat16), wf2_ref[...],
                preferred_element_type=f32) + bf2_ref[...]
    o_ref[...] = o


def kernel(conv1_w, conv1_b, conv2_w, conv2_b, fc1_w, fc1_b,
           bn_scale, bn_shift, fc2_w, fc2_b, x):
    B, Cin, D, H, W = x.shape
    WC = W * Cin
    NB = _NB
    Bp = -(-B // NB) * NB
    if Bp != B:
        x = jnp.pad(x, ((0, Bp - B), (0, 0), (0, 0), (0, 0), (0, 0)))

    xt = jnp.transpose(x.reshape(Bp // NB, NB, Cin, D, H, W),
                       (0, 3, 4, 1, 5, 2)).reshape(Bp // NB, D, H, NB, WC)

    half1 = conv1_b.shape[1]
    half2 = conv2_b.shape[1]
    NOUT = fc2_w.shape[1]

    w1g = conv1_w.reshape(3, 3 * WC, 2 * half1)
    w1g = jnp.pad(w1g, ((0, 0), (0, 128 - 3 * WC), (0, 0))).reshape(384, 2 * half1)
    w1a = w1g[:, :half1]
    w1b = w1g[:, half1:]
    wf1 = fc1_w.reshape(-1, fc1_w.shape[-1])

    grid = Bp // NB
    body = functools.partial(_fused_body, D=D, H=H, WC=WC, NB=NB,
                             half1=half1, half2=half2)

    def full(a):
        return pl.BlockSpec(a.shape, lambda b, _n=a.ndim: (0,) * _n)

    flops = Bp * (2 * (D - 2) * (H - 2) * 384 * 2 * half1
                  + 2 * (S2sq := ((D - 2) // 2 - 2) ** 2) * 9 * half1 * 2 * half2
                  + 2 * wf1.shape[0] * wf1.shape[1] + 2 * 128 * NOUT) // 1
    bytes_accessed = (xt.size * 4 + w1a.size * 2 + w1b.size * 2
                      + conv2_w.size * 2 + wf1.size * 2 + fc2_w.size * 2
                      + Bp * NOUT * 4)

    out = pl.pallas_call(
        body,
        out_shape=jax.ShapeDtypeStruct((Bp, NOUT), jnp.float32),
        grid=(grid,),
        in_specs=[
            pl.BlockSpec((1, D, H, NB, WC), lambda b: (b, 0, 0, 0, 0)),
            full(w1a), full(w1b), full(conv1_b),
            full(conv2_w), full(conv2_b),
            full(wf1), full(fc1_b), full(bn_scale), full(bn_shift),
            full(fc2_w), full(fc2_b),
        ],
        out_specs=pl.BlockSpec((NB, NOUT), lambda b: (b, 0)),
        compiler_params=pltpu.CompilerParams(dimension_semantics=("parallel",)),
        cost_estimate=pl.CostEstimate(flops=flops, transcendentals=0,
                                      bytes_accessed=bytes_accessed),
    )(xt, w1a, w1b, conv1_b, conv2_w, conv2_b,
      wf1, fc1_b, bn_scale, bn_shift, fc2_w, fc2_b)

    return out[:B, :_NUM_CLASSES]


# speedup vs baseline: 1.7903x; 1.7903x over previous
import functools

import jax
import jax.numpy as jnp
from jax.experimental import pallas as pl
from jax.experimental.pallas import tpu as pltpu

_NEG_SLOPE = 0.01
_NUM_CLASSES = 10
_NB = 16


def _lrelu(v):
    return jnp.where(v >= 0.0, v, _NEG_SLOPE * v)


def _fused_body(x_ref, w1a_ref, w1b_ref, b1_ref, w2_ref, b2_ref,
                wf1_ref, bf1_ref, bns_ref, bnt_ref, wf2_ref, bf2_ref,
                o_ref, *, D, H, WC, NB, half1, half2):
    f32 = jnp.float32
    Do1, Ho1 = D - 2, H - 2
    S2 = Do1 // 2
    Do2 = S2 - 2
    Dp2 = Do2 // 2

    xb = x_ref[...].astype(jnp.bfloat16)

    pad = 128 - 3 * WC
    pz = jnp.zeros((D, Ho1, NB, pad), jnp.bfloat16)
    P = jnp.concatenate(
        [xb[:, 0:Ho1], xb[:, 1:1 + Ho1], xb[:, 2:2 + Ho1], pz], axis=-1)

    lhs1 = jnp.concatenate([P[0:Do1], P[1:1 + Do1], P[2:2 + Do1]], axis=-1)
    lhs1 = lhs1.reshape(Do1 * Ho1 * NB, 384)
    q = jnp.maximum(
        jnp.dot(lhs1, w1a_ref[...], preferred_element_type=f32),
        jnp.dot(lhs1, w1b_ref[...], preferred_element_type=f32))
    q = q.reshape(S2, 2, Ho1, NB, half1)
    q = jnp.maximum(q[:, 0], q[:, 1])
    q = q.reshape(S2, S2, 2, NB, half1)
    q = jnp.maximum(q[:, :, 0], q[:, :, 1])
    p1 = _lrelu(q + b1_ref[...]).astype(jnp.bfloat16)

    pieces = [p1[kd:kd + Do2, kh:kh + Do2] for kd in range(3) for kh in range(3)]
    l2 = jnp.concatenate(pieces, axis=-1).reshape(Do2 * Do2 * NB, 9 * half1)
    r2 = jnp.dot(l2, w2_ref[...], preferred_element_type=f32)

    q2 = jnp.maximum(r2[:, :half2], r2[:, half2:])
    q2 = q2.reshape(Dp2, 2, Do2, NB, half2)
    q2 = jnp.maximum(q2[:, 0], q2[:, 1])
    q2 = q2.reshape(Dp2, Dp2, 2, NB, half2)
    q2 = jnp.maximum(q2[:, :, 0], q2[:, :, 1])
    p2 = _lrelu(q2 + b2_ref[...]).astype(jnp.bfloat16)

    fz = jnp.concatenate([p2[0, 0], p2[0, 1], p2[1, 0], p2[1, 1]], axis=-1)
    h = jnp.dot(fz, wf1_ref[...], preferred_element_type=f32) + bf1_ref[...]
    h = _lrelu(h) * bns_ref[...] + bnt_ref[...]
    o = jnp.dot(h.astype(jnp.bfloat16), wf2_ref[...],
                preferred_element_type=f32) + bf2_ref[...]
    o_ref[...] = o


def kernel(conv1_w, conv1_b, conv2_w, conv2_b, fc1_w, fc1_b,
           bn_scale, bn_shift, fc2_w, fc2_b, x):
    B, Cin, D, H, W = x.shape
    WC = W * Cin
    NB = _NB
    Bp = -(-B // NB) * NB
    if Bp != B:
        x = jnp.pad(x, ((0, Bp - B), (0, 0), (0, 0), (0, 0), (0, 0)))

    xt = jnp.transpose(x, (2, 3, 0, 4, 1)).reshape(D, H, Bp, WC)

    half1 = conv1_b.shape[1]
    half2 = conv2_b.shape[1]
    NOUT = fc2_w.shape[1]

    w1g = conv1_w.reshape(3, 3 * WC, 2 * half1)
    w1g = jnp.pad(w1g, ((0, 0), (0, 128 - 3 * WC), (0, 0))).reshape(384, 2 * half1)
    w1a = w1g[:, :half1]
    w1b = w1g[:, half1:]
    wf1 = fc1_w.reshape(-1, fc1_w.shape[-1])

    grid = Bp // NB
    body = functools.partial(_fused_body, D=D, H=H, WC=WC, NB=NB,
                             half1=half1, half2=half2)

    def full(a):
        return pl.BlockSpec(a.shape, lambda b, _n=a.ndim: (0,) * _n)

    flops = Bp * (2 * (D - 2) * (H - 2) * 384 * 2 * half1
                  + 2 * (S2sq := ((D - 2) // 2 - 2) ** 2) * 9 * half1 * 2 * half2
                  + 2 * wf1.shape[0] * wf1.shape[1] + 2 * 128 * NOUT) // 1
    bytes_accessed = (xt.size * 4 + w1a.size * 2 + w1b.size * 2
                      + conv2_w.size * 2 + wf1.size * 2 + fc2_w.size * 2
                      + Bp * NOUT * 4)

    out = pl.pallas_call(
        body,
        out_shape=jax.ShapeDtypeStruct((Bp, NOUT), jnp.float32),
        grid=(grid,),
        in_specs=[
            pl.BlockSpec((D, H, NB, WC), lambda b: (0, 0, b, 0)),
            full(w1a), full(w1b), full(conv1_b),
            full(conv2_w), full(conv2_b),
            full(wf1), full(fc1_b), full(bn_scale), full(bn_shift),
            full(fc2_w), full(fc2_b),
        ],
        out_specs=pl.BlockSpec((NB, NOUT), lambda b: (b, 0)),
        compiler_params=pltpu.CompilerParams(dimension_semantics=("parallel",)),
        cost_estimate=pl.CostEstimate(flops=flops, transcendentals=0,
                                      bytes_accessed=bytes_accessed),
    )(xt, w1a, w1b, conv1_b, conv2_w, conv2_b,
      wf1, fc1_b, bn_scale, bn_shift, fc2_w, fc2_b)

    return out[:B, :_NUM_CLASSES]


# speedup vs baseline: 1.9228x; 1.0740x over previous
import functools

import jax
import jax.numpy as jnp
from jax.experimental import pallas as pl
from jax.experimental.pallas import tpu as pltpu

_NEG_SLOPE = 0.01
_NUM_CLASSES = 10
_NB = 32


def _lrelu(v):
    return jnp.where(v >= 0.0, v, _NEG_SLOPE * v)


def _fused_body(x_ref, w1a_ref, w1b_ref, b1_ref, w2_ref, b2_ref,
                wf1_ref, bf1_ref, bns_ref, bnt_ref, wf2_ref, bf2_ref,
                o_ref, *, D, H, WC, NB, half1, half2):
    f32 = jnp.float32
    Do1, Ho1 = D - 2, H - 2
    S2 = Do1 // 2
    Do2 = S2 - 2
    Dp2 = Do2 // 2

    xb = x_ref[...].astype(jnp.bfloat16)

    pad = 128 - 3 * WC
    pz = jnp.zeros((D, Ho1, NB, pad), jnp.bfloat16)
    P = jnp.concatenate(
        [xb[:, 0:Ho1], xb[:, 1:1 + Ho1], xb[:, 2:2 + Ho1], pz], axis=-1)

    lhs1 = jnp.concatenate([P[0:Do1], P[1:1 + Do1], P[2:2 + Do1]], axis=-1)
    lhs1 = lhs1.reshape(Do1 * Ho1 * NB, 384)
    q = jnp.maximum(
        jnp.dot(lhs1, w1a_ref[...], preferred_element_type=f32),
        jnp.dot(lhs1, w1b_ref[...], preferred_element_type=f32))
    q = q.reshape(S2, 2, Ho1, NB, half1)
    q = jnp.maximum(q[:, 0], q[:, 1])
    q = q.reshape(S2, S2, 2, NB, half1)
    q = jnp.maximum(q[:, :, 0], q[:, :, 1])
    p1 = _lrelu(q + b1_ref[...]).astype(jnp.bfloat16)

    pieces = [p1[kd:kd + Do2, kh:kh + Do2] for kd in range(3) for kh in range(3)]
    l2 = jnp.concatenate(pieces, axis=-1).reshape(Do2 * Do2 * NB, 9 * half1)
    r2 = jnp.dot(l2, w2_ref[...], preferred_element_type=f32)

    q2 = jnp.maximum(r2[:, :half2], r2[:, half2:])
    q2 = q2.reshape(Dp2, 2, Do2, NB, half2)
    q2 = jnp.maximum(q2[:, 0], q2[:, 1])
    q2 = q2.reshape(Dp2, Dp2, 2, NB, half2)
    q2 = jnp.maximum(q2[:, :, 0], q2[:, :, 1])
    p2 = _lrelu(q2 + b2_ref[...]).astype(jnp.bfloat16)

    fz = jnp.concatenate([p2[0, 0], p2[0, 1], p2[1, 0], p2[1, 1]], axis=-1)
    h = jnp.dot(fz, wf1_ref[...], preferred_element_type=f32) + bf1_ref[...]
    h = _lrelu(h) * bns_ref[...] + bnt_ref[...]
    o = jnp.dot(h.astype(jnp.bfloat16), wf2_ref[...],
                preferred_element_type=f32) + bf2_ref[...]
    o_ref[...] = o


def kernel(conv1_w, conv1_b, conv2_w, conv2_b, fc1_w, fc1_b,
           bn_scale, bn_shift, fc2_w, fc2_b, x):
    B, Cin, D, H, W = x.shape
    WC = W * Cin
    NB = _NB
    Bp = -(-B // NB) * NB
    if Bp != B:
        x = jnp.pad(x, ((0, Bp - B), (0, 0), (0, 0), (0, 0), (0, 0)))

    xt = jnp.transpose(x, (2, 3, 0, 4, 1)).reshape(D, H, Bp, WC)

    half1 = conv1_b.shape[1]
    half2 = conv2_b.shape[1]
    NOUT = fc2_w.shape[1]

    w1g = conv1_w.reshape(3, 3 * WC, 2 * half1)
    w1g = jnp.pad(w1g, ((0, 0), (0, 128 - 3 * WC), (0, 0))).reshape(384, 2 * half1)
    w1a = w1g[:, :half1]
    w1b = w1g[:, half1:]
    wf1 = fc1_w.reshape(-1, fc1_w.shape[-1])

    grid = Bp // NB
    body = functools.partial(_fused_body, D=D, H=H, WC=WC, NB=NB,
                             half1=half1, half2=half2)

    def full(a):
        return pl.BlockSpec(a.shape, lambda b, _n=a.ndim: (0,) * _n)

    flops = Bp * (2 * (D - 2) * (H - 2) * 384 * 2 * half1
                  + 2 * (S2sq := ((D - 2) // 2 - 2) ** 2) * 9 * half1 * 2 * half2
                  + 2 * wf1.shape[0] * wf1.shape[1] + 2 * 128 * NOUT) // 1
    bytes_accessed = (xt.size * 4 + w1a.size * 2 + w1b.size * 2
                      + conv2_w.size * 2 + wf1.size * 2 + fc2_w.size * 2
                      + Bp * NOUT * 4)

    out = pl.pallas_call(
        body,
        out_shape=jax.ShapeDtypeStruct((Bp, NOUT), jnp.float32),
        grid=(grid,),
        in_specs=[
            pl.BlockSpec((D, H, NB, WC), lambda b: (0, 0, b, 0)),
            full(w1a), full(w1b), full(conv1_b),
            full(conv2_w), full(conv2_b),
            full(wf1), full(fc1_b), full(bn_scale), full(bn_shift),
            full(fc2_w), full(fc2_b),
        ],
        out_specs=pl.BlockSpec((NB, NOUT), lambda b: (b, 0)),
        compiler_params=pltpu.CompilerParams(dimension_semantics=("parallel",)),
        cost_estimate=pl.CostEstimate(flops=flops, transcendentals=0,
                                      bytes_accessed=bytes_accessed),
    )(xt, w1a, w1b, conv1_b, conv2_w, conv2_b,
      wf1, fc1_b, bn_scale, bn_shift, fc2_w, fc2_b)

    return out[:B, :_NUM_CLASSES]


# speedup vs baseline: 1.9864x; 1.0331x over previous
import functools

import jax
import jax.numpy as jnp
from jax.experimental import pallas as pl
from jax.experimental.pallas import tpu as pltpu

_NEG_SLOPE = 0.01
_NUM_CLASSES = 10
_NB = 64


def _lrelu(v):
    return jnp.where(v >= 0.0, v, _NEG_SLOPE * v)


def _fused_body(x_ref, w1a_ref, w1b_ref, b1_ref, w2_ref, b2_ref,
                wf1_ref, bf1_ref, bns_ref, bnt_ref, wf2_ref, bf2_ref,
                o_ref, *, D, H, WC, NB, half1, half2):
    f32 = jnp.float32
    Do1, Ho1 = D - 2, H - 2
    S2 = Do1 // 2
    Do2 = S2 - 2
    Dp2 = Do2 // 2

    xb = x_ref[...].astype(jnp.bfloat16)

    pad = 128 - 3 * WC
    pz = jnp.zeros((D, Ho1, NB, pad), jnp.bfloat16)
    P = jnp.concatenate(
        [xb[:, 0:Ho1], xb[:, 1:1 + Ho1], xb[:, 2:2 + Ho1], pz], axis=-1)

    lhs1 = jnp.concatenate([P[0:Do1], P[1:1 + Do1], P[2:2 + Do1]], axis=-1)
    lhs1 = lhs1.reshape(Do1 * Ho1 * NB, 384)
    q = jnp.maximum(
        jnp.dot(lhs1, w1a_ref[...], preferred_element_type=f32),
        jnp.dot(lhs1, w1b_ref[...], preferred_element_type=f32))
    q = q.reshape(S2, 2, Ho1, NB, half1)
    q = jnp.maximum(q[:, 0], q[:, 1])
    q = q.reshape(S2, S2, 2, NB, half1)
    q = jnp.maximum(q[:, :, 0], q[:, :, 1])
    p1 = _lrelu(q + b1_ref[...]).astype(jnp.bfloat16)

    pieces = [p1[kd:kd + Do2, kh:kh + Do2] for kd in range(3) for kh in range(3)]
    l2 = jnp.concatenate(pieces, axis=-1).reshape(Do2 * Do2 * NB, 9 * half1)
    r2 = jnp.dot(l2, w2_ref[...], preferred_element_type=f32)

    q2 = jnp.maximum(r2[:, :half2], r2[:, half2:])
    q2 = q2.reshape(Dp2, 2, Do2, NB, half2)
    q2 = jnp.maximum(q2[:, 0], q2[:, 1])
    q2 = q2.reshape(Dp2, Dp2, 2, NB, half2)
    q2 = jnp.maximum(q2[:, :, 0], q2[:, :, 1])
    p2 = _lrelu(q2 + b2_ref[...]).astype(jnp.bfloat16)

    fz = jnp.concatenate([p2[0, 0], p2[0, 1], p2[1, 0], p2[1, 1]], axis=-1)
    h = jnp.dot(fz, wf1_ref[...], preferred_element_type=f32) + bf1_ref[...]
    h = _lrelu(h) * bns_ref[...] + bnt_ref[...]
    o = jnp.dot(h.astype(jnp.bfloat16), wf2_ref[...],
                preferred_element_type=f32) + bf2_ref[...]
    o_ref[...] = o


def kernel(conv1_w, conv1_b, conv2_w, conv2_b, fc1_w, fc1_b,
           bn_scale, bn_shift, fc2_w, fc2_b, x):
    B, Cin, D, H, W = x.shape
    WC = W * Cin
    NB = _NB
    Bp = -(-B // NB) * NB
    if Bp != B:
        x = jnp.pad(x, ((0, Bp - B), (0, 0), (0, 0), (0, 0), (0, 0)))

    xt = jnp.transpose(x, (2, 3, 0, 4, 1)).reshape(D, H, Bp, WC)

    half1 = conv1_b.shape[1]
    half2 = conv2_b.shape[1]
    NOUT = fc2_w.shape[1]

    w1g = conv1_w.reshape(3, 3 * WC, 2 * half1)
    w1g = jnp.pad(w1g, ((0, 0), (0, 128 - 3 * WC), (0, 0))).reshape(384, 2 * half1)
    w1a = w1g[:, :half1]
    w1b = w1g[:, half1:]
    wf1 = fc1_w.reshape(-1, fc1_w.shape[-1])

    grid = Bp // NB
    body = functools.partial(_fused_body, D=D, H=H, WC=WC, NB=NB,
                             half1=half1, half2=half2)

    def full(a):
        return pl.BlockSpec(a.shape, lambda b, _n=a.ndim: (0,) * _n)

    flops = Bp * (2 * (D - 2) * (H - 2) * 384 * 2 * half1
                  + 2 * (S2sq := ((D - 2) // 2 - 2) ** 2) * 9 * half1 * 2 * half2
                  + 2 * wf1.shape[0] * wf1.shape[1] + 2 * 128 * NOUT) // 1
    bytes_accessed = (xt.size * 4 + w1a.size * 2 + w1b.size * 2
                      + conv2_w.size * 2 + wf1.size * 2 + fc2_w.size * 2
                      + Bp * NOUT * 4)

    out = pl.pallas_call(
        body,
        out_shape=jax.ShapeDtypeStruct((Bp, NOUT), jnp.float32),
        grid=(grid,),
        in_specs=[
            pl.BlockSpec((D, H, NB, WC), lambda b: (0, 0, b, 0)),
            full(w1a), full(w1b), full(conv1_b),
            full(conv2_w), full(conv2_b),
            full(wf1), full(fc1_b), full(bn_scale), full(bn_shift),
            full(fc2_w), full(fc2_b),
        ],
        out_specs=pl.BlockSpec((NB, NOUT), lambda b: (b, 0)),
        compiler_params=pltpu.CompilerParams(dimension_semantics=("parallel",)),
        cost_estimate=pl.CostEstimate(flops=flops, transcendentals=0,
                                      bytes_accessed=bytes_accessed),
    )(xt, w1a, w1b, conv1_b, conv2_w, conv2_b,
      wf1, fc1_b, bn_scale, bn_shift, fc2_w, fc2_b)

    return out[:B, :_NUM_CLASSES]


# speedup vs baseline: 2.2714x; 1.1435x over previous
import functools

import jax
import jax.numpy as jnp
from jax.experimental import pallas as pl
from jax.experimental.pallas import tpu as pltpu

_NEG_SLOPE = 0.01
_NUM_CLASSES = 10
_NB = 64


def _lrelu(v):
    return jnp.where(v >= 0.0, v, _NEG_SLOPE * v)


def _fused_body(x_ref, w1a_ref, w1b_ref, b1_ref, w2_ref, b2_ref,
                wf1_ref, bf1_ref, bns_ref, bnt_ref, wf2_ref, bf2_ref,
                o_ref, *, D, H, WC, NB, half1, half2):
    f32 = jnp.float32
    Do1, Ho1 = D - 2, H - 2
    S2 = Do1 // 2
    Do2 = S2 - 2
    Dp2 = Do2 // 2

    xb = x_ref[...]

    pad = 128 - 3 * WC
    pz = jnp.zeros((D, Ho1, NB, pad), jnp.bfloat16)
    P = jnp.concatenate(
        [xb[:, 0:Ho1], xb[:, 1:1 + Ho1], xb[:, 2:2 + Ho1], pz], axis=-1)

    lhs1 = jnp.concatenate([P[0:Do1], P[1:1 + Do1], P[2:2 + Do1]], axis=-1)
    lhs1 = lhs1.reshape(Do1 * Ho1 * NB, 384)
    q = jnp.maximum(
        jnp.dot(lhs1, w1a_ref[...], preferred_element_type=f32),
        jnp.dot(lhs1, w1b_ref[...], preferred_element_type=f32))
    q = q.reshape(S2, 2, Ho1, NB, half1)
    q = jnp.maximum(q[:, 0], q[:, 1])
    q = q.reshape(S2, S2, 2, NB, half1)
    q = jnp.maximum(q[:, :, 0], q[:, :, 1])
    p1 = _lrelu(q + b1_ref[...]).astype(jnp.bfloat16)

    pieces = [p1[kd:kd + Do2, kh:kh + Do2] for kd in range(3) for kh in range(3)]
    l2 = jnp.concatenate(pieces, axis=-1).reshape(Do2 * Do2 * NB, 9 * half1)
    r2 = jnp.dot(l2, w2_ref[...], preferred_element_type=f32)

    q2 = jnp.maximum(r2[:, :half2], r2[:, half2:])
    q2 = q2.reshape(Dp2, 2, Do2, NB, half2)
    q2 = jnp.maximum(q2[:, 0], q2[:, 1])
    q2 = q2.reshape(Dp2, Dp2, 2, NB, half2)
    q2 = jnp.maximum(q2[:, :, 0], q2[:, :, 1])
    p2 = _lrelu(q2 + b2_ref[...]).astype(jnp.bfloat16)

    fz = jnp.concatenate([p2[0, 0], p2[0, 1], p2[1, 0], p2[1, 1]], axis=-1)
    h = jnp.dot(fz, wf1_ref[...], preferred_element_type=f32) + bf1_ref[...]
    h = _lrelu(h) * bns_ref[...] + bnt_ref[...]
    o = jnp.dot(h.astype(jnp.bfloat16), wf2_ref[...],
                preferred_element_type=f32) + bf2_ref[...]
    o_ref[...] = o


def kernel(conv1_w, conv1_b, conv2_w, conv2_b, fc1_w, fc1_b,
           bn_scale, bn_shift, fc2_w, fc2_b, x):
    B, Cin, D, H, W = x.shape
    WC = W * Cin
    NB = _NB
    Bp = -(-B // NB) * NB
    if Bp != B:
        x = jnp.pad(x, ((0, Bp - B), (0, 0), (0, 0), (0, 0), (0, 0)))

    xt = jnp.transpose(x.astype(jnp.bfloat16), (2, 3, 0, 4, 1)).reshape(D, H, Bp, WC)

    half1 = conv1_b.shape[1]
    half2 = conv2_b.shape[1]
    NOUT = fc2_w.shape[1]

    w1g = conv1_w.reshape(3, 3 * WC, 2 * half1)
    w1g = jnp.pad(w1g, ((0, 0), (0, 128 - 3 * WC), (0, 0))).reshape(384, 2 * half1)
    w1a = w1g[:, :half1]
    w1b = w1g[:, half1:]
    wf1 = fc1_w.reshape(-1, fc1_w.shape[-1])

    grid = Bp // NB
    body = functools.partial(_fused_body, D=D, H=H, WC=WC, NB=NB,
                             half1=half1, half2=half2)

    def full(a):
        return pl.BlockSpec(a.shape, lambda b, _n=a.ndim: (0,) * _n)

    flops = Bp * (2 * (D - 2) * (H - 2) * 384 * 2 * half1
                  + 2 * (S2sq := ((D - 2) // 2 - 2) ** 2) * 9 * half1 * 2 * half2
                  + 2 * wf1.shape[0] * wf1.shape[1] + 2 * 128 * NOUT) // 1
    bytes_accessed = (xt.size * 2 + w1a.size * 2 + w1b.size * 2
                      + conv2_w.size * 2 + wf1.size * 2 + fc2_w.size * 2
                      + Bp * NOUT * 4)

    out = pl.pallas_call(
        body,
        out_shape=jax.ShapeDtypeStruct((Bp, NOUT), jnp.float32),
        grid=(grid,),
        in_specs=[
            pl.BlockSpec((D, H, NB, WC), lambda b: (0, 0, b, 0)),
            full(w1a), full(w1b), full(conv1_b),
            full(conv2_w), full(conv2_b),
            full(wf1), full(fc1_b), full(bn_scale), full(bn_shift),
            full(fc2_w), full(fc2_b),
        ],
        out_specs=pl.BlockSpec((NB, NOUT), lambda b: (b, 0)),
        compiler_params=pltpu.CompilerParams(dimension_semantics=("parallel",)),
        cost_estimate=pl.CostEstimate(flops=flops, transcendentals=0,
                                      bytes_accessed=bytes_accessed),
    )(xt, w1a, w1b, conv1_b, conv2_w, conv2_b,
      wf1, fc1_b, bn_scale, bn_shift, fc2_w, fc2_b)

    return out[:B, :_NUM_CLASSES]
